# Initial kernel scaffold; baseline (speedup 1.0000x reference)
import functools

import jax
import jax.numpy as jnp
from jax import lax
from jax.experimental import pallas as pl
from jax.experimental.pallas import tpu as pltpu
from jax.experimental.pallas import tpu_sc as plsc

TS = 256


def _make_sc_gather(vocab, dim, n_idx):
    info = plsc.get_sparse_core_info()
    nc, ns = info.num_cores, info.num_subcores
    nw = nc * ns
    per_w = n_idx // nw
    ch = min(32, per_w)
    chunks = per_w // ch
    mesh = plsc.VectorSubcoreMesh(core_axis_name="c", subcore_axis_name="s")

    @functools.partial(
        pl.kernel,
        mesh=mesh,
        out_type=jax.ShapeDtypeStruct((n_idx, dim), jnp.float32),
        scratch_types=[
            pltpu.VMEM((ch,), jnp.int32),
            pltpu.VMEM((ch, dim), jnp.float32),
            pltpu.SemaphoreType.DMA,
        ],
    )
    def gather(table_hbm, idx_hbm, out_hbm, idx_v, rows_v, sem):
        wid = lax.axis_index("s") * nc + lax.axis_index("c")
        for c in range(chunks):
            base = wid * per_w + c * ch
            pltpu.sync_copy(idx_hbm.at[pl.ds(base, ch)], idx_v)
            pltpu.async_copy(table_hbm.at[idx_v], rows_v, sem).wait()
            pltpu.sync_copy(rows_v, out_hbm.at[pl.ds(base, ch)])

    return gather


def _proj_body(img_ref, aud_ref, wi_ref, bi_ref, wa_ref, ba_ref, out_ref):
    out_ref[...] = (
        jnp.dot(img_ref[...], wi_ref[...], preferred_element_type=jnp.float32)
        + jnp.dot(aud_ref[...], wa_ref[...], preferred_element_type=jnp.float32)
        + bi_ref[...]
        + ba_ref[...]
    )


def _moe_body(text_ref, ma_ref, wg_ref, bg_ref, wx_ref, be_ref, out_ref):
    x = text_ref[0] + ma_ref[...]
    logits = jnp.dot(x, wg_ref[...], preferred_element_type=jnp.float32)
    logits = logits + bg_ref[...]
    m = jnp.max(logits, axis=-1, keepdims=True)
    e = jnp.exp(logits - m)
    gate = e / jnp.sum(e, axis=-1, keepdims=True)
    xb = x.astype(jnp.bfloat16)
    acc = jnp.dot(gate, be_ref[...], preferred_element_type=jnp.float32)
    for n in range(wx_ref.shape[0]):
        mm = jnp.dot(xb, wx_ref[n], preferred_element_type=jnp.float32)
        acc = acc + gate[:, n : n + 1] * mm
    out_ref[0] = acc


def kernel(text_input, image_input, audio_input, emb_table, W_img, b_img,
           W_aud, b_aud, W_gate, b_gate, W_exp, b_exp):
    bsz, seq = text_input.shape
    vocab, emb = emb_table.shape
    nexp = W_exp.shape[0]

    idx = text_input.reshape(-1).astype(jnp.int32)
    text = _make_sc_gather(vocab, emb, bsz * seq)(emb_table, idx)
    text = text.reshape(bsz, seq, emb)

    ma = pl.pallas_call(
        _proj_body,
        out_shape=jax.ShapeDtypeStruct((bsz, emb), jnp.float32),
    )(image_input, audio_input, W_img, b_img.reshape(1, emb),
      W_aud, b_aud.reshape(1, emb))

    wx = W_exp.astype(jnp.bfloat16)
    out = pl.pallas_call(
        _moe_body,
        grid=(bsz, seq // TS),
        in_specs=[
            pl.BlockSpec((1, TS, emb), lambda b, s: (b, s, 0)),
            pl.BlockSpec((1, emb), lambda b, s: (b, 0)),
            pl.BlockSpec((emb, nexp), lambda b, s: (0, 0)),
            pl.BlockSpec((1, nexp), lambda b, s: (0, 0)),
            pl.BlockSpec((nexp, emb, emb), lambda b, s: (0, 0, 0)),
            pl.BlockSpec((nexp, emb), lambda b, s: (0, 0)),
        ],
        out_specs=pl.BlockSpec((1, TS, emb), lambda b, s: (b, s, 0)),
        out_shape=jax.ShapeDtypeStruct((bsz, seq, emb), jnp.float32),
    )(text, ma, W_gate, b_gate.reshape(1, nexp), wx, b_exp)
    return out


# speedup vs baseline: 2.0326x; 2.0326x over previous
import functools

import jax
import jax.numpy as jnp
from jax import lax
from jax.experimental import pallas as pl
from jax.experimental.pallas import tpu as pltpu
from jax.experimental.pallas import tpu_sc as plsc

TS = 256


def _make_sc_gather(vocab, dim, n_idx):
    info = plsc.get_sparse_core_info()
    nc, ns = info.num_cores, info.num_subcores
    nw = nc * ns
    per_w = n_idx // nw
    ch = min(32, per_w)
    chunks = per_w // ch
    mesh = plsc.VectorSubcoreMesh(core_axis_name="c", subcore_axis_name="s")

    @functools.partial(
        pl.kernel,
        mesh=mesh,
        out_type=jax.ShapeDtypeStruct((n_idx, dim), jnp.float32),
        scratch_types=[
            pltpu.VMEM((ch,), jnp.int32),
            pltpu.VMEM((ch, dim), jnp.float32),
            pltpu.SemaphoreType.DMA,
        ],
    )
    def gather(table_hbm, idx_hbm, out_hbm, idx_v, rows_v, sem):
        wid = lax.axis_index("s") * nc + lax.axis_index("c")
        for c in range(chunks):
            base = wid * per_w + c * ch
            pltpu.sync_copy(idx_hbm.at[pl.ds(base, ch)], idx_v)
            pltpu.async_copy(table_hbm.at[idx_v], rows_v, sem).wait()
            pltpu.sync_copy(rows_v, out_hbm.at[pl.ds(base, ch)])

    return gather


def _proj_body(img_ref, aud_ref, wi_ref, bi_ref, wa_ref, ba_ref, out_ref):
    out_ref[...] = (
        jnp.dot(img_ref[...], wi_ref[...], preferred_element_type=jnp.float32)
        + jnp.dot(aud_ref[...], wa_ref[...], preferred_element_type=jnp.float32)
        + bi_ref[...]
        + ba_ref[...]
    )


def _moe_body(text_ref, ma_ref, wg_ref, bg_ref, wx_ref, be_ref, out_ref):
    x = text_ref[0] + ma_ref[0]
    logits = jnp.dot(x, wg_ref[...], preferred_element_type=jnp.float32)
    logits = logits + bg_ref[...]
    m = jnp.max(logits, axis=-1, keepdims=True)
    e = jnp.exp(logits - m)
    gate = e / jnp.sum(e, axis=-1, keepdims=True)
    xb = x.astype(jnp.bfloat16)
    acc = jnp.dot(gate, be_ref[...], preferred_element_type=jnp.float32)
    for n in range(wx_ref.shape[0]):
        mm = jnp.dot(xb, wx_ref[n], preferred_element_type=jnp.float32)
        acc = acc + gate[:, n : n + 1] * mm
    out_ref[0] = acc


def kernel(text_input, image_input, audio_input, emb_table, W_img, b_img,
           W_aud, b_aud, W_gate, b_gate, W_exp, b_exp):
    bsz, seq = text_input.shape
    vocab, emb = emb_table.shape
    nexp = W_exp.shape[0]

    idx = text_input.reshape(-1).astype(jnp.int32)
    text = _make_sc_gather(vocab, emb, bsz * seq)(emb_table, idx)
    text = text.reshape(bsz, seq, emb)

    ma = pl.pallas_call(
        _proj_body,
        out_shape=jax.ShapeDtypeStruct((bsz, emb), jnp.float32),
    )(image_input, audio_input, W_img, b_img.reshape(1, emb),
      W_aud, b_aud.reshape(1, emb))
    ma = ma.reshape(bsz, 1, emb)

    wx = W_exp.astype(jnp.bfloat16)
    out = pl.pallas_call(
        _moe_body,
        grid=(bsz, seq // TS),
        in_specs=[
            pl.BlockSpec((1, TS, emb), lambda b, s: (b, s, 0)),
            pl.BlockSpec((1, 1, emb), lambda b, s: (b, 0, 0)),
            pl.BlockSpec((emb, nexp), lambda b, s: (0, 0)),
            pl.BlockSpec((1, nexp), lambda b, s: (0, 0)),
            pl.BlockSpec((nexp, emb, emb), lambda b, s: (0, 0, 0)),
            pl.BlockSpec((nexp, emb), lambda b, s: (0, 0)),
        ],
        out_specs=pl.BlockSpec((1, TS, emb), lambda b, s: (b, s, 0)),
        out_shape=jax.ShapeDtypeStruct((bsz, seq, emb), jnp.float32),
    )(text, ma, W_gate, b_gate.reshape(1, nexp), wx, b_exp)
    return out


# speedup vs baseline: 2.0535x; 1.0103x over previous
import functools

import jax
import jax.numpy as jnp
from jax import lax
from jax.experimental import pallas as pl
from jax.experimental.pallas import tpu as pltpu
from jax.experimental.pallas import tpu_sc as plsc

TS = 512


def _make_sc_gather(vocab, dim, n_idx):
    info = plsc.get_sparse_core_info()
    nc, ns = info.num_cores, info.num_subcores
    nw = nc * ns
    per_w = n_idx // nw
    ch = min(32, per_w)
    chunks = per_w // ch
    mesh = plsc.VectorSubcoreMesh(core_axis_name="c", subcore_axis_name="s")

    @functools.partial(
        pl.kernel,
        mesh=mesh,
        out_type=jax.ShapeDtypeStruct((n_idx, dim), jnp.float32),
        scratch_types=[
            pltpu.VMEM((ch,), jnp.int32),
            pltpu.VMEM((ch, dim), jnp.float32),
            pltpu.SemaphoreType.DMA,
        ],
    )
    def gather(table_hbm, idx_hbm, out_hbm, idx_v, rows_v, sem):
        wid = lax.axis_index("s") * nc + lax.axis_index("c")
        for c in range(chunks):
            base = wid * per_w + c * ch
            pltpu.sync_copy(idx_hbm.at[pl.ds(base, ch)], idx_v)
            pltpu.async_copy(table_hbm.at[idx_v], rows_v, sem).wait()
            pltpu.sync_copy(rows_v, out_hbm.at[pl.ds(base, ch)])

    return gather


def _proj_body(img_ref, aud_ref, wi_ref, bi_ref, wa_ref, ba_ref, out_ref):
    out_ref[...] = (
        jnp.dot(img_ref[...], wi_ref[...], preferred_element_type=jnp.float32)
        + jnp.dot(aud_ref[...], wa_ref[...], preferred_element_type=jnp.float32)
        + bi_ref[...]
        + ba_ref[...]
    )


def _moe_body(text_ref, ma_ref, wg_ref, bg_ref, wx_ref, be_ref, out_ref):
    x = text_ref[0] + ma_ref[0]
    logits = jnp.dot(x, wg_ref[...], preferred_element_type=jnp.float32)
    logits = logits + bg_ref[...]
    m = jnp.max(logits, axis=-1, keepdims=True)
    e = jnp.exp(logits - m)
    gate = e / jnp.sum(e, axis=-1, keepdims=True)
    xb = x.astype(jnp.bfloat16)
    acc = jnp.dot(gate, be_ref[...], preferred_element_type=jnp.float32)
    for n in range(wx_ref.shape[0]):
        mm = jnp.dot(xb, wx_ref[n], preferred_element_type=jnp.float32)
        acc = acc + gate[:, n : n + 1] * mm
    out_ref[0] = acc


def kernel(text_input, image_input, audio_input, emb_table, W_img, b_img,
           W_aud, b_aud, W_gate, b_gate, W_exp, b_exp):
    bsz, seq = text_input.shape
    vocab, emb = emb_table.shape
    nexp = W_exp.shape[0]

    idx = text_input.reshape(-1).astype(jnp.int32)
    text = _make_sc_gather(vocab, emb, bsz * seq)(emb_table, idx)
    text = text.reshape(bsz, seq, emb)

    ma = pl.pallas_call(
        _proj_body,
        out_shape=jax.ShapeDtypeStruct((bsz, emb), jnp.float32),
    )(image_input, audio_input, W_img, b_img.reshape(1, emb),
      W_aud, b_aud.reshape(1, emb))
    ma = ma.reshape(bsz, 1, emb)

    wx = W_exp.astype(jnp.bfloat16)
    out = pl.pallas_call(
        _moe_body,
        grid=(bsz, seq // TS),
        in_specs=[
            pl.BlockSpec((1, TS, emb), lambda b, s: (b, s, 0)),
            pl.BlockSpec((1, 1, emb), lambda b, s: (b, 0, 0)),
            pl.BlockSpec((emb, nexp), lambda b, s: (0, 0)),
            pl.BlockSpec((1, nexp), lambda b, s: (0, 0)),
            pl.BlockSpec((nexp, emb, emb), lambda b, s: (0, 0, 0)),
            pl.BlockSpec((nexp, emb), lambda b, s: (0, 0)),
        ],
        out_specs=pl.BlockSpec((1, TS, emb), lambda b, s: (b, s, 0)),
        out_shape=jax.ShapeDtypeStruct((bsz, seq, emb), jnp.float32),
    )(text, ma, W_gate, b_gate.reshape(1, nexp), wx, b_exp)
    return out


# speedup vs baseline: 2.1596x; 1.0517x over previous
import functools

import jax
import jax.numpy as jnp
from jax import lax
from jax.experimental import pallas as pl
from jax.experimental.pallas import tpu as pltpu
from jax.experimental.pallas import tpu_sc as plsc

TS = 512


def _make_sc_gather(vocab, dim, n_idx):
    info = plsc.get_sparse_core_info()
    nc, ns = info.num_cores, info.num_subcores
    nw = nc * ns
    per_w = n_idx // nw
    ch = min(32, per_w)
    chunks = per_w // ch
    mesh = plsc.VectorSubcoreMesh(core_axis_name="c", subcore_axis_name="s")

    @functools.partial(
        pl.kernel,
        mesh=mesh,
        out_type=jax.ShapeDtypeStruct((n_idx, dim), jnp.float32),
        scratch_types=[
            pltpu.VMEM((ch,), jnp.int32),
            pltpu.VMEM((ch, dim), jnp.float32),
            pltpu.SemaphoreType.DMA,
        ],
    )
    def gather(table_hbm, idx_hbm, out_hbm, idx_v, rows_v, sem):
        wid = lax.axis_index("s") * nc + lax.axis_index("c")
        for c in range(chunks):
            base = wid * per_w + c * ch
            pltpu.sync_copy(idx_hbm.at[pl.ds(base, ch)], idx_v)
            pltpu.async_copy(table_hbm.at[idx_v], rows_v, sem).wait()
            pltpu.sync_copy(rows_v, out_hbm.at[pl.ds(base, ch)])

    return gather


def _proj_body(img_ref, aud_ref, wi_ref, bi_ref, wa_ref, ba_ref, out_ref):
    out_ref[...] = (
        jnp.dot(img_ref[...], wi_ref[...], preferred_element_type=jnp.float32)
        + jnp.dot(aud_ref[...], wa_ref[...], preferred_element_type=jnp.float32)
        + bi_ref[...]
        + ba_ref[...]
    )


def _moe_body(text_ref, ma_ref, wg_ref, bg_ref, wx_ref, be_ref, out_ref,
              wxs_ref):
    @pl.when((pl.program_id(0) == 0) & (pl.program_id(1) == 0))
    def _cast_weights_once():
        for n in range(wx_ref.shape[0]):
            wxs_ref[n] = wx_ref[n].astype(jnp.bfloat16)

    x = text_ref[0] + ma_ref[0]
    logits = jnp.dot(x, wg_ref[...], preferred_element_type=jnp.float32)
    logits = logits + bg_ref[...]
    m = jnp.max(logits, axis=-1, keepdims=True)
    e = jnp.exp(logits - m)
    gate = e / jnp.sum(e, axis=-1, keepdims=True)
    xb = x.astype(jnp.bfloat16)
    acc = jnp.dot(gate, be_ref[...], preferred_element_type=jnp.float32)
    for n in range(wx_ref.shape[0]):
        mm = jnp.dot(xb, wxs_ref[n], preferred_element_type=jnp.float32)
        acc = acc + gate[:, n : n + 1] * mm
    out_ref[0] = acc


def kernel(text_input, image_input, audio_input, emb_table, W_img, b_img,
           W_aud, b_aud, W_gate, b_gate, W_exp, b_exp):
    bsz, seq = text_input.shape
    vocab, emb = emb_table.shape
    nexp = W_exp.shape[0]

    idx = text_input.reshape(-1).astype(jnp.int32)
    text = _make_sc_gather(vocab, emb, bsz * seq)(emb_table, idx)
    text = text.reshape(bsz, seq, emb)

    ma = pl.pallas_call(
        _proj_body,
        out_shape=jax.ShapeDtypeStruct((bsz, emb), jnp.float32),
    )(image_input, audio_input, W_img, b_img.reshape(1, emb),
      W_aud, b_aud.reshape(1, emb))
    ma = ma.reshape(bsz, 1, emb)

    out = pl.pallas_call(
        _moe_body,
        grid=(bsz, seq // TS),
        scratch_shapes=[pltpu.VMEM((nexp, emb, emb), jnp.bfloat16)],
        compiler_params=pltpu.CompilerParams(
            vmem_limit_bytes=100 * 1024 * 1024,
        ),
        in_specs=[
            pl.BlockSpec((1, TS, emb), lambda b, s: (b, s, 0)),
            pl.BlockSpec((1, 1, emb), lambda b, s: (b, 0, 0)),
            pl.BlockSpec((emb, nexp), lambda b, s: (0, 0)),
            pl.BlockSpec((1, nexp), lambda b, s: (0, 0)),
            pl.BlockSpec((nexp, emb, emb), lambda b, s: (0, 0, 0)),
            pl.BlockSpec((nexp, emb), lambda b, s: (0, 0)),
        ],
        out_specs=pl.BlockSpec((1, TS, emb), lambda b, s: (b, s, 0)),
        out_shape=jax.ShapeDtypeStruct((bsz, seq, emb), jnp.float32),
    )(text, ma, W_gate, b_gate.reshape(1, nexp), W_exp, b_exp)
    return out
